# baseline (speedup 1.0000x reference)
import functools

import jax
import jax.numpy as jnp
from jax import lax
from jax.experimental import pallas as pl
from jax.experimental.pallas import tpu as pltpu
from jax.experimental.pallas import tpu_sc as plsc

_NC = 2
_NS = 16
_NW = _NC * _NS


def _sc_segment_sum(h, src, dst, zeros_nd):
    n, d = h.shape
    e = src.shape[0]
    ch = 80
    assert e % (_NW * ch) == 0
    epw = e // _NW
    nch = epw // ch
    rps = (n // _NS) // 8 * 8
    tail = n - _NS * rps
    assert tail % 8 == 0

    mesh = plsc.VectorSubcoreMesh(
        core_axis_name="c", subcore_axis_name="s",
        num_cores=_NC, num_subcores=_NS)

    scratch = [
        pltpu.VMEM((epw,), jnp.int32),
        pltpu.VMEM((epw,), jnp.int32),
        pltpu.VMEM((ch, d), jnp.float32),
        pltpu.VMEM((ch, d), jnp.float32),
        pltpu.VMEM((ch, d), jnp.float32),
        pltpu.VMEM_SHARED((n, d), jnp.float32),
        pltpu.SemaphoreType.DMA,
        pltpu.SemaphoreType.DMA,
        pltpu.SemaphoreType.DMA,
        pltpu.SemaphoreType.DMA,
        pltpu.SemaphoreType.DMA,
        pltpu.SemaphoreType.DMA,
    ]

    @functools.partial(
        pl.kernel,
        out_type=[jax.ShapeDtypeStruct((n, d), jnp.float32),
                  jax.ShapeDtypeStruct((n, d), jnp.float32)],
        mesh=mesh,
        scratch_types=scratch,
    )
    def seg(h_hbm, src_hbm, dst_hbm, z_hbm, out0_hbm, out1_hbm,
            srcb, dstb, rows0, rows1, rows2, agg_sh,
            gsem0, gsem1, gsem2, ssem0, ssem1, ssem2):
        c = lax.axis_index("c")
        s = lax.axis_index("s")
        wid = c * _NS + s
        rows = (rows0, rows1, rows2)
        gsems = (gsem0, gsem1, gsem2)
        ssems = (ssem0, ssem1, ssem2)

        base_e = wid * epw
        pltpu.async_copy(src_hbm.at[pl.ds(base_e, epw)], srcb, ssem0)
        pltpu.async_copy(dst_hbm.at[pl.ds(base_e, epw)], dstb, ssem1)
        pltpu.async_copy(z_hbm.at[pl.ds(s * rps, rps)],
                         agg_sh.at[pl.ds(s * rps, rps)], ssem2)
        if tail:
            @pl.when(s == _NS - 1)
            def _():
                pltpu.async_copy(z_hbm.at[pl.ds(_NS * rps, tail)],
                                 agg_sh.at[pl.ds(_NS * rps, tail)], ssem2)

        def gather(j, k):
            pltpu.async_copy(h_hbm.at[srcb.at[pl.ds(j * ch, ch)]],
                             rows[k], gsems[k])

        def gwait(k):
            pltpu.make_async_copy(h_hbm.at[pl.ds(0, ch)], rows[k],
                                  gsems[k]).wait()

        def scatter(j, k):
            pltpu.async_copy(rows[k], agg_sh.at[dstb.at[pl.ds(j * ch, ch)]],
                             ssems[k], add=True)

        def swait(k):
            pltpu.make_async_copy(h_hbm.at[pl.ds(0, ch)], rows[k],
                                  ssems[k]).wait()

        pltpu.make_async_copy(src_hbm.at[pl.ds(0, epw)], srcb, ssem0).wait()
        gather(0, 0)
        gather(1, 1)
        pltpu.make_async_copy(dst_hbm.at[pl.ds(0, epw)], dstb, ssem1).wait()
        pltpu.make_async_copy(z_hbm.at[pl.ds(0, rps)],
                              agg_sh.at[pl.ds(s * rps, rps)], ssem2).wait()
        if tail:
            @pl.when(s == _NS - 1)
            def _():
                pltpu.make_async_copy(z_hbm.at[pl.ds(0, tail)],
                                      agg_sh.at[pl.ds(_NS * rps, tail)],
                                      ssem2).wait()
        plsc.subcore_barrier()

        def triple(p, carry):
            for k in range(3):
                j = 3 * p + k
                k2 = (k + 2) % 3

                @pl.when(j < nch)
                def _():
                    gwait(k)
                    scatter(j, k)

                    @pl.when(j >= 1)
                    def _():
                        swait(k2)

                    @pl.when(j + 2 < nch)
                    def _():
                        gather(j + 2, k2)
            return carry

        lax.fori_loop(0, (nch + 2) // 3, triple, 0)
        swait((nch - 1) % 3)

        plsc.subcore_barrier()
        for ci, o_hbm in enumerate((out0_hbm, out1_hbm)):
            @pl.when(c == ci)
            def _():
                pltpu.sync_copy(agg_sh.at[pl.ds(s * rps, rps)],
                                o_hbm.at[pl.ds(s * rps, rps)])
                if tail:
                    @pl.when(s == _NS - 1)
                    def _():
                        pltpu.sync_copy(agg_sh.at[pl.ds(_NS * rps, tail)],
                                        o_hbm.at[pl.ds(_NS * rps, tail)])

    return seg(h, src, dst, zeros_nd)


_BLK = 2000


def _tc_linear(x, w, b, edge_index):
    n, d = x.shape
    hh = w.shape[1]
    e = edge_index.shape[1]
    nb = n // _BLK
    eb = e // nb
    assert n % _BLK == 0 and e % nb == 0

    def body(x_ref, w_ref, b_ref, ei_ref, o_ref, z_ref, src_ref, dst_ref):
        o_ref[...] = (jnp.dot(x_ref[...], w_ref[...],
                              preferred_element_type=jnp.float32)
                      + b_ref[...])
        z_ref[...] = jnp.zeros_like(z_ref)

        @pl.when(pl.program_id(0) == 0)
        def _():
            src_ref[...] = ei_ref[0]
            dst_ref[...] = ei_ref[1]

    return pl.pallas_call(
        body,
        grid=(nb,),
        in_specs=[pl.BlockSpec((_BLK, d), lambda i: (i, 0)),
                  pl.BlockSpec((d, hh), lambda i: (0, 0)),
                  pl.BlockSpec((1, hh), lambda i: (0, 0)),
                  pl.BlockSpec((2, e), lambda i: (0, 0))],
        out_specs=[pl.BlockSpec((_BLK, hh), lambda i: (i, 0)),
                   pl.BlockSpec((_BLK, hh), lambda i: (i, 0)),
                   pl.BlockSpec((e,), lambda i: (0,)),
                   pl.BlockSpec((e,), lambda i: (0,))],
        out_shape=[jax.ShapeDtypeStruct((n, hh), jnp.float32),
                   jax.ShapeDtypeStruct((n, hh), jnp.float32),
                   jax.ShapeDtypeStruct((e,), jnp.int32),
                   jax.ShapeDtypeStruct((e,), jnp.int32)],
    )(x, w, b.reshape(1, hh), edge_index)


def _tc_gin_layer(h, a0, a1, eps, w1, b1, g, be, w2, b2, relu_out):
    n, d = h.shape
    hh = w1.shape[1]
    nb = n // _BLK
    inv_n = 1.0 / n

    def body(eps_ref, h_ref, a0_ref, a1_ref, w1_ref, b1_ref,
             g_ref, be_ref, w2_ref, b2_ref, o_ref, z_scr, st_scr):
        p = pl.program_id(0)
        i = pl.program_id(1)

        @pl.when(p == 0)
        def _():
            z = (1.0 + eps_ref[0, 0]) * h_ref[...] + a0_ref[...] + a1_ref[...]
            z1 = (jnp.dot(z, w1_ref[...], preferred_element_type=jnp.float32)
                  + b1_ref[...])
            z_scr[pl.ds(i * _BLK, _BLK), :] = z1
            stacked = jnp.concatenate([z1, z1 * z1], axis=1)
            ones_row = jnp.ones((8, _BLK), jnp.float32)
            red = jnp.dot(ones_row, stacked,
                          preferred_element_type=jnp.float32)
            part = jnp.reshape(red[0], (2, hh))

            @pl.when(i == 0)
            def _():
                st_scr[...] = jnp.zeros_like(st_scr)

            st_scr[...] += part

        @pl.when(p == 1)
        def _():
            m = st_scr[0:1, :] * inv_n
            var = st_scr[1:2, :] * inv_n - m * m
            scale = lax.rsqrt(var + 1e-5) * g_ref[...]
            zn = (z_scr[pl.ds(i * _BLK, _BLK), :] - m) * scale + be_ref[...]
            zn = jnp.maximum(zn, 0.0)
            o = (jnp.dot(zn, w2_ref[...], preferred_element_type=jnp.float32)
                 + b2_ref[...])
            if relu_out:
                o = jnp.maximum(o, 0.0)
            o_ref[...] = o

    blk_i = lambda p, i: (jnp.where(p == 0, i, 0), 0)
    fixed = lambda p, i: (0, 0)
    return pl.pallas_call(
        body,
        grid=(2, nb),
        in_specs=[pl.BlockSpec(memory_space=pltpu.SMEM),
                  pl.BlockSpec((_BLK, d), blk_i),
                  pl.BlockSpec((_BLK, d), blk_i),
                  pl.BlockSpec((_BLK, d), blk_i),
                  pl.BlockSpec((d, hh), fixed),
                  pl.BlockSpec((1, hh), fixed),
                  pl.BlockSpec((1, hh), fixed),
                  pl.BlockSpec((1, hh), fixed),
                  pl.BlockSpec((hh, d), fixed),
                  pl.BlockSpec((1, d), fixed)],
        out_specs=pl.BlockSpec((_BLK, d), lambda p, i: (jnp.where(p == 1, i, 0), 0)),
        out_shape=jax.ShapeDtypeStruct((n, d), jnp.float32),
        scratch_shapes=[pltpu.VMEM((n, hh), jnp.float32),
                        pltpu.VMEM((2, hh), jnp.float32)],
    )(eps.reshape(1, 1), h, a0, a1, w1, b1.reshape(1, hh),
      g.reshape(1, hh), be.reshape(1, hh), w2, b2.reshape(1, d))


def kernel(x, edge_index, W_lin, b_lin,
           eps0, W1_0, b1_0, g0, be0, W2_0, b2_0,
           eps1, W1_1, b1_1, g1, be1, W2_1, b2_1):
    h, zeros_nd, src, dst = _tc_linear(x, W_lin, b_lin, edge_index)

    a0, a1 = _sc_segment_sum(h, src, dst, zeros_nd)
    h = _tc_gin_layer(h, a0, a1, eps0, W1_0, b1_0,
                      g0, be0, W2_0, b2_0, relu_out=True)

    a0, a1 = _sc_segment_sum(h, src, dst, zeros_nd)
    out = _tc_gin_layer(h, a0, a1, eps1, W1_1, b1_1,
                        g1, be1, W2_1, b2_1, relu_out=False)
    return out


# speedup vs baseline: 1.0096x; 1.0096x over previous
import functools

import jax
import jax.numpy as jnp
from jax import lax
from jax.experimental import pallas as pl
from jax.experimental.pallas import tpu as pltpu
from jax.experimental.pallas import tpu_sc as plsc

_NC = 2
_NS = 16
_NW = _NC * _NS


def _sc_segment_sum(h, src, dst, zeros_nd):
    n, d = h.shape
    e = src.shape[0]
    ch = 80
    assert e % (_NW * ch) == 0
    epw = e // _NW
    nch = epw // ch
    rps = (n // _NS) // 8 * 8
    tail = n - _NS * rps
    assert tail % 8 == 0

    mesh = plsc.VectorSubcoreMesh(
        core_axis_name="c", subcore_axis_name="s",
        num_cores=_NC, num_subcores=_NS)

    scratch = [
        pltpu.VMEM((epw,), jnp.int32),
        pltpu.VMEM((epw,), jnp.int32),
        pltpu.VMEM((ch, d), jnp.float32),
        pltpu.VMEM((ch, d), jnp.float32),
        pltpu.VMEM((ch, d), jnp.float32),
        pltpu.VMEM_SHARED((n, d), jnp.float32),
        pltpu.SemaphoreType.DMA,
        pltpu.SemaphoreType.DMA,
        pltpu.SemaphoreType.DMA,
        pltpu.SemaphoreType.DMA,
        pltpu.SemaphoreType.DMA,
        pltpu.SemaphoreType.DMA,
    ]

    @functools.partial(
        pl.kernel,
        out_type=[jax.ShapeDtypeStruct((n, d), jnp.float32),
                  jax.ShapeDtypeStruct((n, d), jnp.float32)],
        mesh=mesh,
        scratch_types=scratch,
    )
    def seg(h_hbm, src_hbm, dst_hbm, z_hbm, out0_hbm, out1_hbm,
            srcb, dstb, rows0, rows1, rows2, agg_sh,
            gsem0, gsem1, gsem2, ssem0, ssem1, ssem2):
        c = lax.axis_index("c")
        s = lax.axis_index("s")
        wid = c * _NS + s
        rows = (rows0, rows1, rows2)
        gsems = (gsem0, gsem1, gsem2)
        ssems = (ssem0, ssem1, ssem2)

        base_e = wid * epw
        pltpu.async_copy(src_hbm.at[pl.ds(base_e, epw)], srcb, ssem0)
        pltpu.async_copy(dst_hbm.at[pl.ds(base_e, epw)], dstb, ssem1)
        pltpu.async_copy(z_hbm.at[pl.ds(s * rps, rps)],
                         agg_sh.at[pl.ds(s * rps, rps)], ssem2)
        if tail:
            @pl.when(s == _NS - 1)
            def _():
                pltpu.async_copy(z_hbm.at[pl.ds(_NS * rps, tail)],
                                 agg_sh.at[pl.ds(_NS * rps, tail)], ssem2)

        def gather(j, k):
            pltpu.async_copy(h_hbm.at[srcb.at[pl.ds(j * ch, ch)]],
                             rows[k], gsems[k])

        def gwait(k):
            pltpu.make_async_copy(h_hbm.at[pl.ds(0, ch)], rows[k],
                                  gsems[k]).wait()

        def scatter(j, k):
            pltpu.async_copy(rows[k], agg_sh.at[dstb.at[pl.ds(j * ch, ch)]],
                             ssems[k], add=True)

        def swait(k):
            pltpu.make_async_copy(h_hbm.at[pl.ds(0, ch)], rows[k],
                                  ssems[k]).wait()

        pltpu.make_async_copy(src_hbm.at[pl.ds(0, epw)], srcb, ssem0).wait()
        gather(0, 0)
        gather(1, 1)
        pltpu.make_async_copy(dst_hbm.at[pl.ds(0, epw)], dstb, ssem1).wait()
        pltpu.make_async_copy(z_hbm.at[pl.ds(0, rps)],
                              agg_sh.at[pl.ds(s * rps, rps)], ssem2).wait()
        if tail:
            @pl.when(s == _NS - 1)
            def _():
                pltpu.make_async_copy(z_hbm.at[pl.ds(0, tail)],
                                      agg_sh.at[pl.ds(_NS * rps, tail)],
                                      ssem2).wait()
        plsc.subcore_barrier()

        def triple(p, carry):
            for k in range(3):
                j = 3 * p + k
                k2 = (k + 2) % 3

                @pl.when(j < nch)
                def _():
                    gwait(k)
                    scatter(j, k)

                    @pl.when(j >= 1)
                    def _():
                        swait(k2)

                    @pl.when(j + 2 < nch)
                    def _():
                        gather(j + 2, k2)
            return carry

        lax.fori_loop(0, (nch + 2) // 3, triple, 0)
        swait((nch - 1) % 3)

        plsc.subcore_barrier()
        for ci, o_hbm in enumerate((out0_hbm, out1_hbm)):
            @pl.when(c == ci)
            def _():
                pltpu.sync_copy(agg_sh.at[pl.ds(s * rps, rps)],
                                o_hbm.at[pl.ds(s * rps, rps)])
                if tail:
                    @pl.when(s == _NS - 1)
                    def _():
                        pltpu.sync_copy(agg_sh.at[pl.ds(_NS * rps, tail)],
                                        o_hbm.at[pl.ds(_NS * rps, tail)])

    return seg(h, src, dst, zeros_nd)


_BLK = 2000


def _tc_linear(x, w, b, edge_index):
    n, d = x.shape
    hh = w.shape[1]
    e = edge_index.shape[1]
    nb = n // _BLK
    eb = e // nb
    assert n % _BLK == 0 and e % nb == 0

    def body(x_ref, w_ref, b_ref, ei_ref, o_ref, z_ref, src_ref, dst_ref):
        o_ref[...] = (jnp.dot(x_ref[...], w_ref[...],
                              preferred_element_type=jnp.float32)
                      + b_ref[...])
        z_ref[...] = jnp.zeros_like(z_ref)

        @pl.when(pl.program_id(0) == 0)
        def _():
            src_ref[...] = ei_ref[0]
            dst_ref[...] = ei_ref[1]

    return pl.pallas_call(
        body,
        grid=(nb,),
        in_specs=[pl.BlockSpec((_BLK, d), lambda i: (i, 0)),
                  pl.BlockSpec((d, hh), lambda i: (0, 0)),
                  pl.BlockSpec((1, hh), lambda i: (0, 0)),
                  pl.BlockSpec((2, e), lambda i: (0, 0))],
        out_specs=[pl.BlockSpec((_BLK, hh), lambda i: (i, 0)),
                   pl.BlockSpec((_BLK, hh), lambda i: (i, 0)),
                   pl.BlockSpec((e,), lambda i: (0,)),
                   pl.BlockSpec((e,), lambda i: (0,))],
        out_shape=[jax.ShapeDtypeStruct((n, hh), jnp.float32),
                   jax.ShapeDtypeStruct((n, hh), jnp.float32),
                   jax.ShapeDtypeStruct((e,), jnp.int32),
                   jax.ShapeDtypeStruct((e,), jnp.int32)],
    )(x, w, b.reshape(1, hh), edge_index)


def _tc_gin_layer(h, a0, a1, eps, w1, b1, g, be, w2, b2, relu_out):
    n, d = h.shape
    hh = w1.shape[1]
    nb = n // _BLK
    inv_n = 1.0 / n

    def body(eps_ref, h_ref, a0_ref, a1_ref, w1_ref, b1_ref,
             g_ref, be_ref, w2_ref, b2_ref, o_ref, z_scr, st_scr):
        p = pl.program_id(0)
        i = pl.program_id(1)

        @pl.when(p == 0)
        def _():
            z = (1.0 + eps_ref[0, 0]) * h_ref[...] + a0_ref[...] + a1_ref[...]
            z1 = (jnp.dot(z, w1_ref[...], preferred_element_type=jnp.float32)
                  + b1_ref[...])
            z_scr[pl.ds(i * _BLK, _BLK), :] = z1
            ps = jnp.sum(z1, axis=0, keepdims=True)
            pq = jnp.sum(z1 * z1, axis=0, keepdims=True)
            part = jnp.concatenate([ps, pq], axis=0)

            @pl.when(i == 0)
            def _():
                st_scr[...] = jnp.zeros_like(st_scr)

            st_scr[...] += part

        @pl.when(p == 1)
        def _():
            m = st_scr[0:1, :] * inv_n
            var = st_scr[1:2, :] * inv_n - m * m
            scale = lax.rsqrt(var + 1e-5) * g_ref[...]
            zn = (z_scr[pl.ds(i * _BLK, _BLK), :] - m) * scale + be_ref[...]
            zn = jnp.maximum(zn, 0.0)
            o = (jnp.dot(zn, w2_ref[...], preferred_element_type=jnp.float32)
                 + b2_ref[...])
            if relu_out:
                o = jnp.maximum(o, 0.0)
            o_ref[...] = o

    blk_i = lambda p, i: (jnp.where(p == 0, i, 0), 0)
    fixed = lambda p, i: (0, 0)
    return pl.pallas_call(
        body,
        grid=(2, nb),
        in_specs=[pl.BlockSpec(memory_space=pltpu.SMEM),
                  pl.BlockSpec((_BLK, d), blk_i),
                  pl.BlockSpec((_BLK, d), blk_i),
                  pl.BlockSpec((_BLK, d), blk_i),
                  pl.BlockSpec((d, hh), fixed),
                  pl.BlockSpec((1, hh), fixed),
                  pl.BlockSpec((1, hh), fixed),
                  pl.BlockSpec((1, hh), fixed),
                  pl.BlockSpec((hh, d), fixed),
                  pl.BlockSpec((1, d), fixed)],
        out_specs=pl.BlockSpec((_BLK, d), lambda p, i: (jnp.where(p == 1, i, 0), 0)),
        out_shape=jax.ShapeDtypeStruct((n, d), jnp.float32),
        scratch_shapes=[pltpu.VMEM((n, hh), jnp.float32),
                        pltpu.VMEM((2, hh), jnp.float32)],
    )(eps.reshape(1, 1), h, a0, a1, w1, b1.reshape(1, hh),
      g.reshape(1, hh), be.reshape(1, hh), w2, b2.reshape(1, d))


def kernel(x, edge_index, W_lin, b_lin,
           eps0, W1_0, b1_0, g0, be0, W2_0, b2_0,
           eps1, W1_1, b1_1, g1, be1, W2_1, b2_1):
    h, zeros_nd, src, dst = _tc_linear(x, W_lin, b_lin, edge_index)

    a0, a1 = _sc_segment_sum(h, src, dst, zeros_nd)
    h = _tc_gin_layer(h, a0, a1, eps0, W1_0, b1_0,
                      g0, be0, W2_0, b2_0, relu_out=True)

    a0, a1 = _sc_segment_sum(h, src, dst, zeros_nd)
    out = _tc_gin_layer(h, a0, a1, eps1, W1_1, b1_1,
                        g1, be1, W2_1, b2_1, relu_out=False)
    return out
